# baseline (speedup 1.0000x reference)
import jax
import jax.numpy as jnp
from jax import lax
from jax.experimental import pallas as pl
from jax.experimental.pallas import tpu as pltpu
from jax.experimental.pallas import tpu_sc as plsc

NC = 2
NS = 16
L = 16
NW = NC * NS

B = 16384
K = 32
V = 1000000
BPW = B // NW
NG = BPW // L
NBUF = 8


def _mf_body(uid_hbm, iid_hbm, uwT_hbm, iwT_hbm, out_hbm,
             uidx_v, iidx_v, uch, ich, dummy_v, out_v, sems):
    wid = lax.axis_index("s") * NC + lax.axis_index("c")
    base = wid * BPW

    lanes = jnp.arange(L, dtype=jnp.int32)
    zerosf = jnp.zeros((L,), jnp.float32)

    pltpu.sync_copy(uid_hbm.at[pl.ds(base, BPW)], uidx_v)
    pltpu.sync_copy(iid_hbm.at[pl.ds(base, BPW)], iidx_v)

    def zero(i, _):
        out_v[pl.ds(i * L, L)] = zerosf
        return 0
    lax.fori_loop(0, NG, zero, 0, unroll=4)

    def chunk_of(idx_vec, j):
        return (idx_vec[j] // 128) * 128

    def start(ju, ji, b):
        cu = pltpu.async_copy(uwT_hbm.at[:, pl.ds(ju, 128)], uch[b],
                              sems.at[b, 0])
        ci = pltpu.async_copy(iwT_hbm.at[:, pl.ds(ji, 128)], ich[b],
                              sems.at[b, 1])
        return cu, ci

    def wait(ju, ji, b):
        pltpu.make_async_copy(uwT_hbm.at[:, pl.ds(ju, 128)], uch[b],
                              sems.at[b, 0]).wait()
        pltpu.make_async_copy(iwT_hbm.at[:, pl.ds(ji, 128)], ich[b],
                              sems.at[b, 1]).wait()

    def compute(cu, ci, n, b):
        ucol = jnp.full((L,), cu, jnp.int32)
        icol = jnp.full((L,), ci, jnp.int32)
        u0 = plsc.load_gather(dummy_v, [lanes, ucol])
        u1 = plsc.load_gather(dummy_v, [lanes + L, ucol])
        v0 = plsc.load_gather(dummy_v, [lanes, icol])
        v1 = plsc.load_gather(dummy_v, [lanes + L, icol])
        s = u0 * v0 + u1 * v1
        plsc.addupdate_scatter(out_v, [jnp.full((L,), n, jnp.int32)], s)

    u_cur = uidx_v[pl.ds(0, L)]
    i_cur = iidx_v[pl.ds(0, L)]
    for b in range(NBUF):
        start(chunk_of(u_cur, b), chunk_of(i_cur, b), b)

    def group(g, carry):
        u_cur, i_cur = carry
        gp1 = jnp.minimum(g + 1, NG - 1)
        u_nxt = uidx_v[pl.ds(gp1 * L, L)]
        i_nxt = iidx_v[pl.ds(gp1 * L, L)]
        n0 = g * L
        for j in range(L):
            b = j % NBUF
            wait(chunk_of(u_cur, j), chunk_of(i_cur, j), b)
            compute(u_cur[j] % 128, i_cur[j] % 128, n0 + j, b)
            jn = j + NBUF
            if jn < L:
                start(chunk_of(u_cur, jn), chunk_of(i_cur, jn), b)
            else:
                @pl.when(g < NG - 1)
                def _():
                    start(chunk_of(u_nxt, jn - L), chunk_of(i_nxt, jn - L), b)
        return (u_nxt, i_nxt)

    lax.fori_loop(0, NG, group, (u_cur, i_cur))

    pltpu.sync_copy(out_v, out_hbm.at[pl.ds(base, BPW)])


@jax.jit
def _mf(uid, iid, uwT, iwT):
    mesh = plsc.VectorSubcoreMesh(
        core_axis_name="c", subcore_axis_name="s",
        num_cores=NC, num_subcores=NS)
    return pl.kernel(
        _mf_body,
        out_type=jax.ShapeDtypeStruct((B,), jnp.float32),
        mesh=mesh,
        compiler_params=pltpu.CompilerParams(
            needs_layout_passes=False, use_tc_tiling_on_sc=True),
        scratch_types=[
            pltpu.VMEM((BPW,), jnp.int32),
            pltpu.VMEM((BPW,), jnp.int32),
            [pltpu.VMEM_SHARED((K, 128), jnp.float32) for _ in range(NBUF)],
            [pltpu.VMEM_SHARED((K, 128), jnp.float32) for _ in range(NBUF)],
            pltpu.VMEM((K, 128), jnp.float32),
            pltpu.VMEM((BPW,), jnp.float32),
            pltpu.SemaphoreType.DMA((NBUF, 2)),
        ],
    )(uid, iid, uwT, iwT)


def kernel(train_x, user_weight, item_weight):
    tx = jnp.asarray(train_x, jnp.int32)
    return _mf(tx[:, 0], tx[:, 1], user_weight.T, item_weight.T)


# speedup vs baseline: 1.4540x; 1.4540x over previous
import jax
import jax.numpy as jnp
from jax import lax
from jax.experimental import pallas as pl
from jax.experimental.pallas import tpu as pltpu
from jax.experimental.pallas import tpu_sc as plsc

NC = 2
NS = 16
L = 16
NW = NC * NS

B = 16384
K = 32
V = 1000000
BPW = B // NW
NG = BPW // L
NBUF = 8


def _mf_body(uid_hbm, iid_hbm, uwT_hbm, iwT_hbm, out_hbm,
             uidx_v, iidx_v, uch, ich, out_v, sems):
    wid = lax.axis_index("s") * NC + lax.axis_index("c")
    base = wid * BPW

    lanes = jnp.arange(L, dtype=jnp.int32)
    zerosf = jnp.zeros((L,), jnp.float32)

    pltpu.sync_copy(uid_hbm.at[pl.ds(base, BPW)], uidx_v)
    pltpu.sync_copy(iid_hbm.at[pl.ds(base, BPW)], iidx_v)

    def zero(i, _):
        out_v[pl.ds(i * L, L)] = zerosf
        return 0
    lax.fori_loop(0, NG, zero, 0, unroll=4)

    def chunk_of(idx_vec, j):
        return (idx_vec[j] // 128) * 128

    def start(ju, ji, b):
        cu = pltpu.async_copy(uwT_hbm.at[:, pl.ds(ju, 128)], uch[b],
                              sems.at[b, 0])
        ci = pltpu.async_copy(iwT_hbm.at[:, pl.ds(ji, 128)], ich[b],
                              sems.at[b, 1])
        return cu, ci

    def wait(ju, ji, b):
        pltpu.make_async_copy(uwT_hbm.at[:, pl.ds(ju, 128)], uch[b],
                              sems.at[b, 0]).wait()
        pltpu.make_async_copy(iwT_hbm.at[:, pl.ds(ji, 128)], ich[b],
                              sems.at[b, 1]).wait()

    def compute(cu, ci, n, b):
        ucol = jnp.full((L,), cu, jnp.int32)
        icol = jnp.full((L,), ci, jnp.int32)
        u0 = plsc.load_gather(uch[b], [lanes, ucol])
        u1 = plsc.load_gather(uch[b], [lanes + L, ucol])
        v0 = plsc.load_gather(ich[b], [lanes, icol])
        v1 = plsc.load_gather(ich[b], [lanes + L, icol])
        s = u0 * v0 + u1 * v1
        plsc.addupdate_scatter(out_v, [jnp.full((L,), n, jnp.int32)], s)

    u_cur = uidx_v[pl.ds(0, L)]
    i_cur = iidx_v[pl.ds(0, L)]
    for b in range(NBUF):
        start(chunk_of(u_cur, b), chunk_of(i_cur, b), b)

    def group(g, carry):
        u_cur, i_cur = carry
        gp1 = jnp.minimum(g + 1, NG - 1)
        u_nxt = uidx_v[pl.ds(gp1 * L, L)]
        i_nxt = iidx_v[pl.ds(gp1 * L, L)]
        n0 = g * L
        for j in range(L):
            b = j % NBUF
            wait(chunk_of(u_cur, j), chunk_of(i_cur, j), b)
            compute(u_cur[j] % 128, i_cur[j] % 128, n0 + j, b)
            jn = j + NBUF
            if jn < L:
                start(chunk_of(u_cur, jn), chunk_of(i_cur, jn), b)
            else:
                @pl.when(g < NG - 1)
                def _():
                    start(chunk_of(u_nxt, jn - L), chunk_of(i_nxt, jn - L), b)
        return (u_nxt, i_nxt)

    lax.fori_loop(0, NG, group, (u_cur, i_cur))

    pltpu.sync_copy(out_v, out_hbm.at[pl.ds(base, BPW)])


@jax.jit
def _mf(uid, iid, uwT, iwT):
    mesh = plsc.VectorSubcoreMesh(
        core_axis_name="c", subcore_axis_name="s",
        num_cores=NC, num_subcores=NS)
    return pl.kernel(
        _mf_body,
        out_type=jax.ShapeDtypeStruct((B,), jnp.float32),
        mesh=mesh,
        compiler_params=pltpu.CompilerParams(
            needs_layout_passes=False, use_tc_tiling_on_sc=True),
        scratch_types=[
            pltpu.VMEM((BPW,), jnp.int32),
            pltpu.VMEM((BPW,), jnp.int32),
            [pltpu.VMEM((K, 128), jnp.float32) for _ in range(NBUF)],
            [pltpu.VMEM((K, 128), jnp.float32) for _ in range(NBUF)],
            pltpu.VMEM((BPW,), jnp.float32),
            pltpu.SemaphoreType.DMA((NBUF, 2)),
        ],
    )(uid, iid, uwT, iwT)


def kernel(train_x, user_weight, item_weight):
    tx = jnp.asarray(train_x, jnp.int32)
    return _mf(tx[:, 0], tx[:, 1], user_weight.T, item_weight.T)


# speedup vs baseline: 1.4543x; 1.0002x over previous
import jax
import jax.numpy as jnp
from jax import lax
from jax.experimental import pallas as pl
from jax.experimental.pallas import tpu as pltpu
from jax.experimental.pallas import tpu_sc as plsc

NC = 2
NS = 16
L = 16
NW = NC * NS

B = 16384
K = 32
BPW = B // NW
NG = BPW // L
NBUF = 8


def _mf_body(uid_hbm, iid_hbm, uwT_hbm, iwT_hbm, out_hbm,
             uidx_v, iidx_v, uch, ich, out_v, sems):
    wid = lax.axis_index("s") * NC + lax.axis_index("c")
    base = wid * BPW

    lanes = jnp.arange(L, dtype=jnp.int32)
    zerosf = jnp.zeros((L,), jnp.float32)

    pltpu.sync_copy(uid_hbm.at[pl.ds(base, BPW)], uidx_v)
    pltpu.sync_copy(iid_hbm.at[pl.ds(base, BPW)], iidx_v)

    def zero(i, _):
        out_v[pl.ds(i * L, L)] = zerosf
        return 0
    lax.fori_loop(0, NG, zero, 0, unroll=4)

    def chunk_of(idx_vec, j):
        return (idx_vec[j] // 128) * 128

    def start(ju, ji, b):
        cu = pltpu.async_copy(uwT_hbm.at[:, pl.ds(ju, 128)], uch[b],
                              sems.at[b, 0])
        ci = pltpu.async_copy(iwT_hbm.at[:, pl.ds(ji, 128)], ich[b],
                              sems.at[b, 1])
        return cu, ci

    def wait(ju, ji, b):
        pltpu.make_async_copy(uwT_hbm.at[:, pl.ds(ju, 128)], uch[b],
                              sems.at[b, 0]).wait()
        pltpu.make_async_copy(iwT_hbm.at[:, pl.ds(ji, 128)], ich[b],
                              sems.at[b, 1]).wait()

    def compute(cu, ci, n, b):
        ucol = jnp.full((L,), cu, jnp.int32)
        icol = jnp.full((L,), ci, jnp.int32)
        u0 = plsc.load_gather(uch[b], [lanes, ucol])
        u1 = plsc.load_gather(uch[b], [lanes + L, ucol])
        v0 = plsc.load_gather(ich[b], [lanes, icol])
        v1 = plsc.load_gather(ich[b], [lanes + L, icol])
        s = u0 * v0 + u1 * v1
        plsc.addupdate_scatter(out_v, [jnp.full((L,), n, jnp.int32)], s)

    u_cur = uidx_v[pl.ds(0, L)]
    i_cur = iidx_v[pl.ds(0, L)]
    for b in range(NBUF):
        start(chunk_of(u_cur, b), chunk_of(i_cur, b), b)

    def group(g, carry):
        u_cur, i_cur = carry
        gp1 = jnp.minimum(g + 1, NG - 1)
        u_nxt = uidx_v[pl.ds(gp1 * L, L)]
        i_nxt = iidx_v[pl.ds(gp1 * L, L)]
        n0 = g * L
        for j in range(L):
            b = j % NBUF
            wait(chunk_of(u_cur, j), chunk_of(i_cur, j), b)
            compute(u_cur[j] % 128, i_cur[j] % 128, n0 + j, b)
            jn = j + NBUF
            if jn < L:
                start(chunk_of(u_cur, jn), chunk_of(i_cur, jn), b)
            else:
                @pl.when(g < NG - 1)
                def _():
                    start(chunk_of(u_nxt, jn - L), chunk_of(i_nxt, jn - L), b)
        return (u_nxt, i_nxt)

    lax.fori_loop(0, NG, group, (u_cur, i_cur))

    pltpu.sync_copy(out_v, out_hbm.at[pl.ds(base, BPW)])


@jax.jit
def _mf(uid, iid, uwT, iwT):
    mesh = plsc.VectorSubcoreMesh(
        core_axis_name="c", subcore_axis_name="s",
        num_cores=NC, num_subcores=NS)
    return pl.kernel(
        _mf_body,
        out_type=jax.ShapeDtypeStruct((B,), jnp.float32),
        mesh=mesh,
        compiler_params=pltpu.CompilerParams(
            needs_layout_passes=False, use_tc_tiling_on_sc=True),
        scratch_types=[
            pltpu.VMEM((BPW,), jnp.int32),
            pltpu.VMEM((BPW,), jnp.int32),
            [pltpu.VMEM((K, 128), jnp.float32) for _ in range(NBUF)],
            [pltpu.VMEM((K, 128), jnp.float32) for _ in range(NBUF)],
            pltpu.VMEM((BPW,), jnp.float32),
            pltpu.SemaphoreType.DMA((NBUF, 2)),
        ],
    )(uid, iid, uwT, iwT)


def kernel(train_x, user_weight, item_weight):
    tx = jnp.asarray(train_x, jnp.int32)
    return _mf(tx[:, 0], tx[:, 1], user_weight.T, item_weight.T)
